# baseline (speedup 1.0000x reference)
import functools

import jax
import jax.numpy as jnp
from jax import lax
from jax.experimental import pallas as pl
from jax.experimental.pallas import tpu as pltpu
from jax.experimental.pallas import tpu_sc as plsc

EMBED = 128
BATCH = 16384
NUM_TABLES = 4
NC = 2
NS = 16
NW = NC * NS
BPW = BATCH // NW


def _build():
    mesh = plsc.VectorSubcoreMesh(core_axis_name="c", subcore_axis_name="s")

    @functools.partial(
        pl.kernel,
        mesh=mesh,
        out_type=jax.ShapeDtypeStruct((BATCH, NUM_TABLES * EMBED), jnp.float32),
        scratch_types=[
            pltpu.VMEM((BPW,), jnp.int32),
            pltpu.VMEM((BPW, EMBED), jnp.float32),
            pltpu.SemaphoreType.DMA,
        ],
    )
    def k(g_idx, a_idx, o_idx, z_idx, g_tbl, a_tbl, o_tbl, z_tbl,
          out, idx_v, rows_v, sem):
        wid = lax.axis_index("s") * NC + lax.axis_index("c")
        base = wid * BPW
        pairs = ((g_idx, g_tbl), (a_idx, a_tbl), (o_idx, o_tbl), (z_idx, z_tbl))
        for t, (idx_hbm, tbl_hbm) in enumerate(pairs):
            pltpu.sync_copy(idx_hbm.at[pl.ds(base, BPW)], idx_v)
            pltpu.async_copy(tbl_hbm.at[idx_v], rows_v, sem).wait()
            pltpu.sync_copy(
                rows_v, out.at[pl.ds(base, BPW), pl.ds(t * EMBED, EMBED)])

    return k


_sc_call = _build()


def kernel(gender_idx, age_idx, occupation_idx, area_idx,
           gender_table, age_table, occupation_table, area_table):
    return _sc_call(
        gender_idx.astype(jnp.int32), age_idx.astype(jnp.int32),
        occupation_idx.astype(jnp.int32), area_idx.astype(jnp.int32),
        gender_table, age_table, occupation_table, area_table)
